# baseline (device time: 109444 ns/iter reference)
import jax
import jax.numpy as jnp
from jax import lax
from jax.experimental import pallas as pl
from jax.experimental.pallas import tpu as pltpu

W = 32
TC = 128


def kernel(x, A, B, C):
    Bb, S, D = x.shape
    N = A.shape[1]
    Bh = Bb // 2
    n_chunks = S // TC

    def body(x_ref, a_ref, b_ref, c_ref, out_ref,
             h_ref, bts_ref, cts_ref, xhr_ref, bhr_ref, snd_ref, rcv_ref,
             halo_send_sems, halo_recv_sems,
             out_send_sems, out_recv_sems, ack_sem):
        my_x = lax.axis_index("x")
        my_y = lax.axis_index("y")
        bs = my_x * Bh
        nb = (1 - my_x) * Bh

        xh_rdma = pltpu.make_async_remote_copy(
            src_ref=x_ref.at[pl.ds(bs, Bh), pl.ds(S - W, W), :],
            dst_ref=xhr_ref,
            send_sem=halo_send_sems.at[0], recv_sem=halo_recv_sems.at[0],
            device_id=(my_x, 1 - my_y),
            device_id_type=pl.DeviceIdType.MESH,
        )
        bh_rdma = pltpu.make_async_remote_copy(
            src_ref=b_ref.at[pl.ds(bs, Bh), pl.ds(S - W, W), :],
            dst_ref=bhr_ref,
            send_sem=halo_send_sems.at[1], recv_sem=halo_recv_sems.at[1],
            device_id=(my_x, 1 - my_y),
            device_id_type=pl.DeviceIdType.MESH,
        )

        @pl.when(my_y == 0)
        def _():
            xh_rdma.start()
            bh_rdma.start()

        bts_ref[...] = jnp.swapaxes(b_ref[pl.ds(bs, Bh)], 1, 2)
        cts_ref[...] = jnp.swapaxes(c_ref[pl.ds(bs, Bh)], 1, 2)
        h_ref[...] = jnp.zeros_like(h_ref)
        da = jnp.exp(a_ref[...]).T[None, :, :]

        @pl.when(my_y == 1)
        def _():
            xh_rdma.wait_recv()
            bh_rdma.wait_recv()
            pl.semaphore_signal(
                ack_sem, inc=1,
                device_id=(my_x, 0),
                device_id_type=pl.DeviceIdType.MESH,
            )
            b_all = jnp.swapaxes(bhr_ref[...], 1, 2)
            iota_w = lax.broadcasted_iota(jnp.int32, (Bh, N, W), 2)

            def wstep(k, carry):
                b_t = jnp.sum(jnp.where(iota_w == k, b_all, 0.0),
                              axis=2, keepdims=True)
                x_t = xhr_ref[:, pl.ds(k, 1), :]
                h_ref[...] = h_ref[...] * da + x_t * b_t
                return carry

            lax.fori_loop(0, W, wstep, 0, unroll=8)

        def out_chunk_rdma(c):
            sl = (slice(None), pl.ds(c * TC, TC), slice(None))
            return pltpu.make_async_remote_copy(
                src_ref=snd_ref.at[sl], dst_ref=rcv_ref.at[sl],
                send_sem=out_send_sems.at[c], recv_sem=out_recv_sems.at[c],
                device_id=(1 - my_x, my_y),
                device_id_type=pl.DeviceIdType.MESH,
            )

        def chunk(c, carry):
            b_blk = bts_ref[:, :, pl.ds(c * TC, TC)]
            c_blk = cts_ref[:, :, pl.ds(c * TC, TC)]
            iota = lax.broadcasted_iota(jnp.int32, (Bh, N, TC), 2)

            def step(k, carry2):
                t = c * TC + k
                msk = iota == k
                b_t = jnp.sum(jnp.where(msk, b_blk, 0.0), axis=2,
                              keepdims=True)
                c_t = jnp.sum(jnp.where(msk, c_blk, 0.0), axis=2,
                              keepdims=True)
                x_t = x_ref[pl.ds(bs, Bh), pl.ds(t, 1), :]
                h = h_ref[...] * da + x_t * b_t
                h_ref[...] = h
                out_ref[pl.ds(bs, Bh), pl.ds(t, 1), :] = jnp.sum(
                    h * c_t, axis=1, keepdims=True)
                return carry2

            lax.fori_loop(0, TC, step, 0, unroll=32)
            tsl = pl.ds(c * TC, TC)
            snd_ref[:, tsl, :] = out_ref[pl.ds(bs, Bh), tsl, :].astype(
                jnp.bfloat16)
            out_chunk_rdma(c).start()
            return carry

        lax.fori_loop(0, n_chunks, chunk, 0)

        def drain(c, carry):
            rdma = out_chunk_rdma(c)
            rdma.wait_send()
            rdma.wait_recv()
            tsl = pl.ds(c * TC, TC)
            out_ref[pl.ds(nb, Bh), tsl, :] = rcv_ref[:, tsl, :].astype(
                jnp.float32)
            return carry

        lax.fori_loop(0, n_chunks, drain, 0)

        @pl.when(my_y == 0)
        def _():
            xh_rdma.wait_send()
            bh_rdma.wait_send()
            pl.semaphore_wait(ack_sem, 1)

    return pl.pallas_call(
        body,
        out_shape=jax.ShapeDtypeStruct((Bb, S, D), jnp.float32),
        in_specs=[pl.BlockSpec(memory_space=pltpu.VMEM)] * 4,
        out_specs=pl.BlockSpec(memory_space=pltpu.VMEM),
        scratch_shapes=[
            pltpu.VMEM((Bh, N, D), jnp.float32),
            pltpu.VMEM((Bh, N, S), jnp.float32),
            pltpu.VMEM((Bh, N, S), jnp.float32),
            pltpu.VMEM((Bh, W, D), jnp.float32),
            pltpu.VMEM((Bh, W, N), jnp.float32),
            pltpu.VMEM((Bh, S, D), jnp.bfloat16),
            pltpu.VMEM((Bh, S, D), jnp.bfloat16),
            pltpu.SemaphoreType.DMA((2,)),
            pltpu.SemaphoreType.DMA((2,)),
            pltpu.SemaphoreType.DMA((n_chunks,)),
            pltpu.SemaphoreType.DMA((n_chunks,)),
            pltpu.SemaphoreType.REGULAR,
        ],
    )(x, A, B, C)


# device time: 106953 ns/iter; 1.0233x vs baseline; 1.0233x over previous
import jax
import jax.numpy as jnp
from jax import lax
from jax.experimental import pallas as pl
from jax.experimental.pallas import tpu as pltpu

W = 32
TC = 128


def kernel(x, A, B, C):
    Bb, S, D = x.shape
    N = A.shape[1]
    Bh = Bb // 2
    n_chunks = S // TC

    def body(x_ref, a_ref, b_ref, c_ref, out_ref,
             h_ref, bts_ref, cts_ref, xhr_ref, bhr_ref, snd_ref, rcv_ref,
             halo_send_sems, halo_recv_sems,
             out_send_sems, out_recv_sems, ack_sem):
        my_x = lax.axis_index("x")
        my_y = lax.axis_index("y")
        bs = my_x * Bh
        nb = (1 - my_x) * Bh

        xh_rdma = pltpu.make_async_remote_copy(
            src_ref=x_ref.at[pl.ds(bs, Bh), pl.ds(S - W, W), :],
            dst_ref=xhr_ref,
            send_sem=halo_send_sems.at[0], recv_sem=halo_recv_sems.at[0],
            device_id=(my_x, 1 - my_y),
            device_id_type=pl.DeviceIdType.MESH,
        )
        bh_rdma = pltpu.make_async_remote_copy(
            src_ref=b_ref.at[pl.ds(bs, Bh), pl.ds(S - W, W), :],
            dst_ref=bhr_ref,
            send_sem=halo_send_sems.at[1], recv_sem=halo_recv_sems.at[1],
            device_id=(my_x, 1 - my_y),
            device_id_type=pl.DeviceIdType.MESH,
        )

        @pl.when(my_y == 0)
        def _():
            xh_rdma.start()
            bh_rdma.start()

        bts_ref[...] = jnp.swapaxes(b_ref[pl.ds(bs, Bh)], 1, 2)
        cts_ref[...] = jnp.swapaxes(c_ref[pl.ds(bs, Bh)], 1, 2)
        h_ref[...] = jnp.zeros_like(h_ref)
        da = jnp.exp(a_ref[...]).T[None, :, :]

        @pl.when(my_y == 1)
        def _():
            xh_rdma.wait_recv()
            bh_rdma.wait_recv()
            pl.semaphore_signal(
                ack_sem, inc=1,
                device_id=(my_x, 0),
                device_id_type=pl.DeviceIdType.MESH,
            )
            b_all = jnp.swapaxes(bhr_ref[...], 1, 2)
            iota_w = lax.broadcasted_iota(jnp.int32, (Bh, N, W), 2)

            def wstep(k, carry):
                b_t = jnp.sum(jnp.where(iota_w == k, b_all, 0.0),
                              axis=2, keepdims=True)
                x_t = xhr_ref[:, pl.ds(k, 1), :]
                h_ref[...] = h_ref[...] * da + x_t * b_t
                return carry

            lax.fori_loop(0, W, wstep, 0, unroll=8)

        TH = TC // 2

        def out_half_rdma(hc):
            sl = (slice(None), pl.ds(hc * TH, TH), slice(None))
            return pltpu.make_async_remote_copy(
                src_ref=snd_ref.at[sl], dst_ref=rcv_ref.at[sl],
                send_sem=out_send_sems.at[hc], recv_sem=out_recv_sems.at[hc],
                device_id=(1 - my_x, my_y),
                device_id_type=pl.DeviceIdType.MESH,
            )

        def chunk(c, carry):
            b_blk = bts_ref[:, :, pl.ds(c * TC, TC)]
            c_blk = cts_ref[:, :, pl.ds(c * TC, TC)]
            iota = lax.broadcasted_iota(jnp.int32, (Bh, N, TC), 2)

            def step(k, carry2):
                t = c * TC + k
                msk = iota == k
                b_t = jnp.sum(jnp.where(msk, b_blk, 0.0), axis=2,
                              keepdims=True)
                c_t = jnp.sum(jnp.where(msk, c_blk, 0.0), axis=2,
                              keepdims=True)
                x_t = x_ref[pl.ds(bs, Bh), pl.ds(t, 1), :]
                h = h_ref[...] * da + x_t * b_t
                h_ref[...] = h
                out_ref[pl.ds(bs, Bh), pl.ds(t, 1), :] = jnp.sum(
                    h * c_t, axis=1, keepdims=True)
                return carry2

            def ship_half(hc):
                tsl = pl.ds(hc * TH, TH)
                snd_ref[:, tsl, :] = out_ref[
                    pl.ds(bs, Bh), tsl, :].astype(jnp.bfloat16)
                out_half_rdma(hc).start()

            lax.fori_loop(0, TH, step, 0, unroll=32)
            ship_half(2 * c)
            lax.fori_loop(TH, TC, step, 0, unroll=32)
            ship_half(2 * c + 1)
            return carry

        lax.fori_loop(0, n_chunks, chunk, 0)

        def drain(hc, carry):
            rdma = out_half_rdma(hc)
            rdma.wait_send()
            rdma.wait_recv()
            tsl = pl.ds(hc * TH, TH)
            out_ref[pl.ds(nb, Bh), tsl, :] = rcv_ref[:, tsl, :].astype(
                jnp.float32)
            return carry

        lax.fori_loop(0, 2 * n_chunks, drain, 0)

        @pl.when(my_y == 0)
        def _():
            xh_rdma.wait_send()
            bh_rdma.wait_send()
            pl.semaphore_wait(ack_sem, 1)

    return pl.pallas_call(
        body,
        out_shape=jax.ShapeDtypeStruct((Bb, S, D), jnp.float32),
        in_specs=[pl.BlockSpec(memory_space=pltpu.VMEM)] * 4,
        out_specs=pl.BlockSpec(memory_space=pltpu.VMEM),
        scratch_shapes=[
            pltpu.VMEM((Bh, N, D), jnp.float32),
            pltpu.VMEM((Bh, N, S), jnp.float32),
            pltpu.VMEM((Bh, N, S), jnp.float32),
            pltpu.VMEM((Bh, W, D), jnp.float32),
            pltpu.VMEM((Bh, W, N), jnp.float32),
            pltpu.VMEM((Bh, S, D), jnp.bfloat16),
            pltpu.VMEM((Bh, S, D), jnp.bfloat16),
            pltpu.SemaphoreType.DMA((2,)),
            pltpu.SemaphoreType.DMA((2,)),
            pltpu.SemaphoreType.DMA((2 * n_chunks,)),
            pltpu.SemaphoreType.DMA((2 * n_chunks,)),
            pltpu.SemaphoreType.REGULAR,
        ],
    )(x, A, B, C)
